# baseline (speedup 1.0000x reference)
import functools

import jax
import jax.numpy as jnp
from jax import lax
from jax.experimental import pallas as pl
from jax.experimental.pallas import tpu as pltpu
from jax.experimental.pallas import tpu_sc as plsc

_BATCH = 4096
_SEQ = 200
_D = 64
_TABLE = 201
_KPAD = 208
_BPR = 256
_WPR = _BPR // 4

_INFO = plsc.get_sparse_core_info()
_NC = _INFO.num_cores
_NS = _INFO.num_subcores
_LANES = _INFO.num_lanes
_NW = _NC * _NS
_ROWS = _BATCH // _NW
_GROUPS = _ROWS // _LANES

_mesh = plsc.VectorSubcoreMesh(core_axis_name="c", subcore_axis_name="s")


@functools.partial(
    pl.kernel,
    out_type=jax.ShapeDtypeStruct((_BATCH, _KPAD), jnp.float32),
    mesh=_mesh,
    compiler_params=pltpu.CompilerParams(needs_layout_passes=False),
    scratch_types=[
        pltpu.VMEM((_ROWS * _WPR,), jnp.int32),
        pltpu.VMEM((_ROWS, _KPAD), jnp.float32),
    ],
)
def _histogram(pos_hbm, counts_hbm, pos_v, cnt_v):
    wid = lax.axis_index("s") * _NC + lax.axis_index("c")
    base = wid * _ROWS

    pltpu.sync_copy(pos_hbm.at[pl.ds(base * _WPR, _ROWS * _WPR)], pos_v)

    ones = jnp.ones((_LANES,), jnp.float32)
    zeros = jnp.zeros((_LANES,), jnp.float32)

    _UNROLL = 8

    def row_body(i, _):
        rows = [i * _UNROLL + k for k in range(_UNROLL)]
        for r in rows:
            for u in range(_KPAD // _LANES):
                cnt_v[r, pl.ds(u * _LANES, _LANES)] = zeros
        rvecs = [jnp.full((_LANES,), r, jnp.int32) for r in rows]
        for c in range(_WPR // _LANES):
            ws = [
                pos_v[pl.ds(r * _WPR + c * _LANES, _LANES)] for r in rows
            ]
            for b in range(4):
                for rv, w in zip(rvecs, ws):
                    p = lax.shift_right_logical(w, 8 * b) & 255
                    plsc.addupdate_scatter(
                        cnt_v, [rv, p], ones, mask=p < _TABLE
                    )
        return 0
    lax.fori_loop(0, _ROWS // _UNROLL, row_body, 0)

    pltpu.sync_copy(cnt_v, counts_hbm.at[pl.ds(base, _ROWS), :])


def _tc_body(x_ref, c_ref, pe_ref, o_ref):
    o_ref[...] = x_ref[...] + jnp.dot(
        c_ref[...], pe_ref[...], preferred_element_type=jnp.float32
    )


def kernel(x, positions, pe):
    posp = jnp.pad(
        positions.astype(jnp.int32),
        ((0, 0), (0, _BPR - _SEQ)),
        constant_values=255,
    )
    pos_packed = (
        posp[:, 0:64]
        | (posp[:, 64:128] << 8)
        | (posp[:, 128:192] << 16)
        | (posp[:, 192:256] << 24)
    ).reshape(-1)
    counts = _histogram(pos_packed)
    pe_pad = jnp.concatenate(
        [pe[0], jnp.zeros((_KPAD - _TABLE, _D), pe.dtype)], axis=0
    )

    grid = 16
    rows = _BATCH // grid
    out = pl.pallas_call(
        _tc_body,
        grid=(grid,),
        in_specs=[
            pl.BlockSpec((rows, _D), lambda i: (i, 0)),
            pl.BlockSpec((rows, _KPAD), lambda i: (i, 0)),
            pl.BlockSpec((_KPAD, _D), lambda i: (0, 0)),
        ],
        out_specs=pl.BlockSpec((rows, _D), lambda i: (i, 0)),
        out_shape=jax.ShapeDtypeStruct((_BATCH, _D), jnp.float32),
    )(x, counts, pe_pad)
    return out


# speedup vs baseline: 1.1674x; 1.1674x over previous
import functools

import jax
import jax.numpy as jnp
from jax import lax
from jax.experimental import pallas as pl
from jax.experimental.pallas import tpu as pltpu
from jax.experimental.pallas import tpu_sc as plsc

_BATCH = 4096
_SEQ = 200
_D = 64
_TABLE = 201
_KPAD = 208
_BPR = 256
_WPR = _BPR // 4

_INFO = plsc.get_sparse_core_info()
_NC = _INFO.num_cores
_NS = _INFO.num_subcores
_LANES = _INFO.num_lanes
_NW = _NC * _NS
_ROWS = _BATCH // _NW
_GROUPS = _ROWS // _LANES

_mesh = plsc.VectorSubcoreMesh(core_axis_name="c", subcore_axis_name="s")


@functools.partial(
    pl.kernel,
    out_type=jax.ShapeDtypeStruct((_BATCH, _KPAD), jnp.float32),
    mesh=_mesh,
    compiler_params=pltpu.CompilerParams(needs_layout_passes=False),
    scratch_types=[
        pltpu.VMEM((_ROWS * _WPR,), jnp.int32),
        pltpu.VMEM((_ROWS, _KPAD), jnp.float32),
    ],
)
def _histogram(pos_hbm, counts_hbm, pos_v, cnt_v):
    wid = lax.axis_index("s") * _NC + lax.axis_index("c")
    base = wid * _ROWS

    pltpu.sync_copy(pos_hbm.at[pl.ds(base * _WPR, _ROWS * _WPR)], pos_v)

    ones = jnp.ones((_LANES,), jnp.float32)
    zeros = jnp.zeros((_LANES,), jnp.float32)

    _UNROLL = 8

    def row_body(i, _):
        rows = [i * _UNROLL + k for k in range(_UNROLL)]
        for r in rows:
            for u in range(_KPAD // _LANES):
                cnt_v[r, pl.ds(u * _LANES, _LANES)] = zeros
        rvecs = [jnp.full((_LANES,), r, jnp.int32) for r in rows]
        for c in range(_WPR // _LANES):
            ws = [
                pos_v[pl.ds(r * _WPR + c * _LANES, _LANES)] for r in rows
            ]
            for b in range(4):
                for rv, w in zip(rvecs, ws):
                    p = lax.shift_right_logical(w, 8 * b) & 255
                    plsc.addupdate_scatter(
                        cnt_v, [rv, p], ones, mask=p < _TABLE
                    )
        return 0
    lax.fori_loop(0, _ROWS // _UNROLL, row_body, 0)

    pltpu.sync_copy(cnt_v, counts_hbm.at[pl.ds(base, _ROWS), :])


def _tc_body(x_ref, c_ref, pe_ref, o_ref):
    o_ref[...] = x_ref[...] + jnp.dot(
        c_ref[...], pe_ref[...], preferred_element_type=jnp.float32
    )


def kernel(x, positions, pe):
    posp = jnp.pad(
        positions.astype(jnp.int32),
        ((0, 0), (0, _BPR - _SEQ)),
        constant_values=255,
    )
    pos_packed = (
        posp[:, 0:64]
        | (posp[:, 64:128] << 8)
        | (posp[:, 128:192] << 16)
        | (posp[:, 192:256] << 24)
    ).reshape(-1)
    counts = _histogram(pos_packed)
    pe_pad = jnp.concatenate(
        [pe[0], jnp.zeros((_KPAD - _TABLE, _D), pe.dtype)], axis=0
    )

    grid = 4
    rows = _BATCH // grid
    out = pl.pallas_call(
        _tc_body,
        grid=(grid,),
        in_specs=[
            pl.BlockSpec((rows, _D), lambda i: (i, 0)),
            pl.BlockSpec((rows, _KPAD), lambda i: (i, 0)),
            pl.BlockSpec((_KPAD, _D), lambda i: (0, 0)),
        ],
        out_specs=pl.BlockSpec((rows, _D), lambda i: (i, 0)),
        out_shape=jax.ShapeDtypeStruct((_BATCH, _D), jnp.float32),
    )(x, counts, pe_pad)
    return out


# speedup vs baseline: 1.2080x; 1.0348x over previous
import functools

import jax
import jax.numpy as jnp
from jax import lax
from jax.experimental import pallas as pl
from jax.experimental.pallas import tpu as pltpu
from jax.experimental.pallas import tpu_sc as plsc

_BATCH = 4096
_SEQ = 200
_D = 64
_TABLE = 201
_KPAD = 208
_BPR = 256
_WPR = _BPR // 4

_INFO = plsc.get_sparse_core_info()
_NC = _INFO.num_cores
_NS = _INFO.num_subcores
_LANES = _INFO.num_lanes
_NW = _NC * _NS
_ROWS = _BATCH // _NW
_GROUPS = _ROWS // _LANES

_mesh = plsc.VectorSubcoreMesh(core_axis_name="c", subcore_axis_name="s")


@functools.partial(
    pl.kernel,
    out_type=jax.ShapeDtypeStruct((_BATCH, _KPAD), jnp.float32),
    mesh=_mesh,
    compiler_params=pltpu.CompilerParams(needs_layout_passes=False),
    scratch_types=[
        pltpu.VMEM((_ROWS * _WPR,), jnp.int32),
        pltpu.VMEM((_ROWS, _KPAD), jnp.float32),
    ],
)
def _histogram(pos_hbm, counts_hbm, pos_v, cnt_v):
    wid = lax.axis_index("s") * _NC + lax.axis_index("c")
    base = wid * _ROWS

    pltpu.sync_copy(pos_hbm.at[pl.ds(base * _WPR, _ROWS * _WPR)], pos_v)

    ones = jnp.ones((_LANES,), jnp.float32)
    zeros = jnp.zeros((_LANES,), jnp.float32)

    _UNROLL = 8

    def row_body(i, _):
        rows = [i * _UNROLL + k for k in range(_UNROLL)]
        for r in rows:
            for u in range(_KPAD // _LANES):
                cnt_v[r, pl.ds(u * _LANES, _LANES)] = zeros
        rvecs = [jnp.full((_LANES,), r, jnp.int32) for r in rows]
        for c in range(_WPR // _LANES):
            ws = [
                pos_v[pl.ds(r * _WPR + c * _LANES, _LANES)] for r in rows
            ]
            for b in range(4):
                for rv, w in zip(rvecs, ws):
                    p = lax.shift_right_logical(w, 8 * b) & 255
                    plsc.addupdate_scatter(
                        cnt_v, [rv, p], ones, mask=p < _TABLE
                    )
        return 0
    lax.fori_loop(0, _ROWS // _UNROLL, row_body, 0)

    pltpu.sync_copy(cnt_v, counts_hbm.at[pl.ds(base, _ROWS), :])


def _tc_body(x_ref, c_ref, pe_ref, o_ref):
    o_ref[...] = x_ref[...] + jnp.dot(
        c_ref[...], pe_ref[...], preferred_element_type=jnp.float32
    )


def kernel(x, positions, pe):
    posp = jnp.pad(
        positions.astype(jnp.int32),
        ((0, 0), (0, _BPR - _SEQ)),
        constant_values=255,
    )
    pos_packed = (
        posp[:, 0:64]
        | (posp[:, 64:128] << 8)
        | (posp[:, 128:192] << 16)
        | (posp[:, 192:256] << 24)
    ).reshape(-1)
    counts = _histogram(pos_packed)
    pe_pad = jnp.concatenate(
        [pe[0], jnp.zeros((_KPAD - _TABLE, _D), pe.dtype)], axis=0
    )

    grid = 2
    rows = _BATCH // grid
    out = pl.pallas_call(
        _tc_body,
        grid=(grid,),
        in_specs=[
            pl.BlockSpec((rows, _D), lambda i: (i, 0)),
            pl.BlockSpec((rows, _KPAD), lambda i: (i, 0)),
            pl.BlockSpec((_KPAD, _D), lambda i: (0, 0)),
        ],
        out_specs=pl.BlockSpec((rows, _D), lambda i: (i, 0)),
        out_shape=jax.ShapeDtypeStruct((_BATCH, _D), jnp.float32),
    )(x, counts, pe_pad)
    return out
